# baseline (speedup 1.0000x reference)
import functools

import jax
import jax.numpy as jnp
from jax import lax
from jax.experimental import pallas as pl
from jax.experimental.pallas import tpu as pltpu
from jax.experimental.pallas import tpu_sc as plsc

N = 100000
D = 128
H = 64
C = 16
K = 10
ALPHA = 0.1

NC = 2
NS = 16
NW = NC * NS

CB = 500
NCH = 200

CBC = 10000
NCHC = 10

NPAD = 100352
RPT = NPAD // NS
RPW = NPAD // NW
CRC = RPW // 4
ZR = 392

_mesh = plsc.VectorSubcoreMesh(core_axis_name="c", subcore_axis_name="s")


@functools.partial(
    pl.kernel,
    out_type=jax.ShapeDtypeStruct((NC, NPAD), jnp.float32),
    mesh=_mesh,
    compiler_params=pltpu.CompilerParams(use_tc_tiling_on_sc=False),
    scratch_types=[
        pltpu.VMEM((3, CBC), jnp.int32),
        pltpu.VMEM((CBC,), jnp.float32),
        pltpu.VMEM((RPT,), jnp.float32),
        pltpu.VMEM_SHARED((NPAD,), jnp.float32),
        pltpu.SemaphoreType.DMA,
        pltpu.SemaphoreType.DMA,
        pltpu.SemaphoreType.DMA,
        pltpu.SemaphoreType.DMA,
        pltpu.SemaphoreType.DMA,
    ],
)
def _count(dstp, ones_h, zeros_h, deg_out, idx, ones_v, zb, agg,
           ia, ib, ic, sa, sb):
    c = lax.axis_index("c")
    s = lax.axis_index("s")
    w = c * NS + s
    isem = [ia, ib, ic]
    ssem = [sa, sb]
    pltpu.async_copy(dstp.at[w, 0], idx.at[0], isem[0])
    pltpu.async_copy(dstp.at[w, 1], idx.at[1], isem[1])
    pltpu.sync_copy(zeros_h, zb)
    pltpu.sync_copy(ones_h, ones_v)
    pltpu.sync_copy(zb, agg.at[pl.ds(s * RPT, RPT)])
    plsc.subcore_barrier()

    for ch in range(NCHC):
        pltpu.make_async_copy(dstp.at[0, 0], idx.at[ch % 3],
                              isem[ch % 3]).wait()
        if ch >= 1:
            pltpu.make_async_copy(ones_v, agg.at[pl.ds(0, CBC)],
                                  ssem[(ch - 1) % 2]).wait()
        pltpu.async_copy(ones_v, agg.at[idx.at[ch % 3]], ssem[ch % 2],
                         add=True)
        if ch + 2 < NCHC:
            pltpu.async_copy(dstp.at[w, ch + 2], idx.at[(ch + 2) % 3],
                             isem[(ch + 2) % 3])
    pltpu.make_async_copy(ones_v, agg.at[pl.ds(0, CBC)],
                          ssem[(NCHC - 1) % 2]).wait()
    plsc.subcore_barrier()
    pltpu.sync_copy(agg.at[pl.ds(s * RPT, RPT)],
                    deg_out.at[c, pl.ds(s * RPT, RPT)])


@functools.partial(
    pl.kernel,
    out_type=jax.ShapeDtypeStruct((NC, NPAD, C), jnp.float32),
    mesh=_mesh,
    compiler_params=pltpu.CompilerParams(use_tc_tiling_on_sc=False),
    scratch_types=[
        pltpu.VMEM((4, 2, CB), jnp.int32),
        pltpu.VMEM((3, CB, C), jnp.float32),
        pltpu.VMEM_SHARED((NPAD, C), jnp.float32),
        pltpu.SemaphoreType.DMA,
        pltpu.SemaphoreType.DMA,
        pltpu.SemaphoreType.DMA,
        pltpu.SemaphoreType.DMA,
        pltpu.SemaphoreType.DMA,
        pltpu.SemaphoreType.DMA,
        pltpu.SemaphoreType.DMA,
        pltpu.SemaphoreType.DMA,
        pltpu.SemaphoreType.DMA,
        pltpu.SemaphoreType.DMA,
        pltpu.SemaphoreType.DMA,
    ],
)
def _round(u, srcp, dstp, p_out, idx, rows, agg,
           i0, i1, i2, i3, g0, g1, g2, s0, s1, s2, zs):
    c = lax.axis_index("c")
    s = lax.axis_index("s")
    w = c * NS + s
    isem = [i0, i1, i2, i3]
    gsem = [g0, g1, g2]
    ssem = [s0, s1, s2]

    def i_start(ch, b):
        pltpu.async_copy(srcp.at[w, ch], idx.at[b].at[0], isem[b])
        pltpu.async_copy(dstp.at[w, ch], idx.at[b].at[1], isem[b])

    def i_wait(b):
        pltpu.make_async_copy(srcp.at[0, 0], idx.at[b].at[0], isem[b]).wait()
        pltpu.make_async_copy(dstp.at[0, 0], idx.at[b].at[1], isem[b]).wait()

    def g_start(ib, b):
        pltpu.async_copy(u.at[idx.at[ib].at[0]], rows.at[b], gsem[b])

    def g_wait(b):
        pltpu.make_async_copy(u.at[pl.ds(0, CB)], rows.at[b], gsem[b]).wait()

    def s_start(ib, b):
        pltpu.async_copy(rows.at[b], agg.at[idx.at[ib].at[1]], ssem[b],
                         add=True)

    def s_wait(b):
        pltpu.make_async_copy(rows.at[b], agg.at[pl.ds(0, CB)],
                              ssem[b]).wait()

    i_start(0, 0)
    i_start(1, 1)
    i_start(2, 2)
    i_wait(0)
    g_start(0, 0)
    i_wait(1)
    g_start(1, 1)

    def zfill(j, carry):
        rows[2, j] = jnp.zeros((C,), jnp.float32)
        return carry

    lax.fori_loop(0, CB, zfill, 0)
    nz = RPT // CB
    for j in range(nz):
        pltpu.async_copy(rows.at[2], agg.at[pl.ds(s * RPT + j * CB, CB)], zs)
    rem = RPT - nz * CB
    if rem:
        pltpu.async_copy(rows.at[2].at[pl.ds(0, rem)],
                         agg.at[pl.ds(s * RPT + nz * CB, rem)], zs)
    for j in range(nz):
        pltpu.make_async_copy(rows.at[2], agg.at[pl.ds(0, CB)], zs).wait()
    if rem:
        pltpu.make_async_copy(rows.at[2].at[pl.ds(0, rem)],
                              agg.at[pl.ds(0, rem)], zs).wait()
    plsc.subcore_barrier()

    def step(ch, j):
        r = j % 3
        g_wait(r)
        s_start(j % 4, r)

        @pl.when(ch >= 1)
        def _():
            s_wait((j + 2) % 3)

        i_wait((j + 2) % 4)
        g_start((j + 2) % 4, (j + 2) % 3)

        @pl.when(ch + 3 < NCH)
        def _():
            i_start(ch + 3, (j + 3) % 4)

    def twelve(m, carry):
        for j in range(12):
            step(m * 12 + j, j)
        return carry

    nfull = (NCH - 2) // 12
    lax.fori_loop(0, nfull, twelve, 0)
    for t in range(NCH - 2 - nfull * 12):
        step(nfull * 12 + t, t)

    e = NCH - 2
    g_wait(e % 3)
    s_wait((e + 2) % 3)
    s_start(e % 4, e % 3)
    e = NCH - 1
    g_wait(e % 3)
    s_start(e % 4, e % 3)
    s_wait((NCH - 2) % 3)
    s_wait((NCH - 1) % 3)

    plsc.subcore_barrier()
    pltpu.sync_copy(agg.at[pl.ds(s * RPT, RPT)],
                    p_out.at[c, pl.ds(s * RPT, RPT)])


NTC = 7
CR2 = RPW // NTC


@functools.partial(
    pl.kernel,
    out_type=jax.ShapeDtypeStruct((NPAD, C), jnp.float32),
    mesh=_mesh,
    compiler_params=pltpu.CompilerParams(use_tc_tiling_on_sc=False),
    scratch_types=[
        pltpu.VMEM((2, CR2, C), jnp.float32),
        pltpu.VMEM((2, CR2, C), jnp.float32),
        pltpu.VMEM((2, CR2, C), jnp.float32),
        pltpu.VMEM((2, CR2, C), jnp.float32),
        pltpu.VMEM((2, CR2, C), jnp.float32),
        pltpu.SemaphoreType.DMA,
        pltpu.SemaphoreType.DMA,
        pltpu.SemaphoreType.DMA,
        pltpu.SemaphoreType.DMA,
    ],
)
def _combine(p, u, c1b, c2b, un, bu, b0, b1, bc1, bc2, la, lb, wa, wb):
    c = lax.axis_index("c")
    s = lax.axis_index("s")
    w = c * NS + s
    base = w * RPW
    lsem = [la, lb]
    wsem = [wa, wb]

    def loads(t, b):
        r0 = base + t * CR2
        pltpu.async_copy(u.at[pl.ds(r0, CR2)], bu.at[b], lsem[b])
        pltpu.async_copy(p.at[0, pl.ds(r0, CR2)], b0.at[b], lsem[b])
        pltpu.async_copy(p.at[1, pl.ds(r0, CR2)], b1.at[b], lsem[b])
        pltpu.async_copy(c1b.at[pl.ds(r0, CR2)], bc1.at[b], lsem[b])
        pltpu.async_copy(c2b.at[pl.ds(r0, CR2)], bc2.at[b], lsem[b])

    def loads_wait(b):
        for ref in (bu, b0, b1, bc1, bc2):
            pltpu.make_async_copy(u.at[pl.ds(0, CR2)], ref.at[b],
                                  lsem[b]).wait()

    def store(t, b):
        pltpu.async_copy(bu.at[b], un.at[pl.ds(base + t * CR2, CR2)],
                         wsem[b])

    def store_wait(b):
        pltpu.make_async_copy(bu.at[b], un.at[pl.ds(base, CR2)],
                              wsem[b]).wait()

    loads(0, 0)
    for t in range(NTC):
        b = t % 2
        nb = (t + 1) % 2
        loads_wait(b)
        if t + 1 < NTC:
            if t >= 1:
                store_wait(nb)
            loads(t + 1, nb)

        def row(j, carry):
            for q in range(4):
                i = j * 4 + q
                bu[b, i] = bc1[b, i] * (b0[b, i] + b1[b, i] + bu[b, i]) \
                    + bc2[b, i]
            return carry

        lax.fori_loop(0, CR2 // 4, row, 0)
        store(t, b)
    store_wait((NTC - 2) % 2)
    store_wait((NTC - 1) % 2)


def _proj_body(x_ref, w1_ref, b1_ref, w2_ref, b2_ref, d0_ref, d1_ref,
               u0_ref, c1_ref, c2_ref):
    i = pl.program_id(0)
    xb = x_ref[...]
    h = jnp.maximum(
        jnp.dot(xb, w1_ref[...], preferred_element_type=jnp.float32)
        + b1_ref[...], 0.0)
    zb = (jnp.dot(h, w2_ref[...], preferred_element_type=jnp.float32)
          + b2_ref[...])
    deg = d0_ref[...] + d1_ref[...] + 1.0
    rows = lax.broadcasted_iota(jnp.int32, deg.shape, 0) + i * deg.shape[0]
    m = jnp.where(rows < N, 1.0, 0.0)
    dinv = lax.rsqrt(deg)
    u0 = zb * dinv * m
    u0_ref[...] = u0
    c1_ref[...] = jnp.broadcast_to((1.0 - ALPHA) / deg * m, zb.shape)
    c2_ref[...] = ALPHA * u0


def _proj(x, w1t, b1r, w2t, b2r, d0, d1):
    br = 4096
    grid = ((NPAD + br - 1) // br,)
    return pl.pallas_call(
        _proj_body,
        grid=grid,
        in_specs=[
            pl.BlockSpec((br, D), lambda i: (i, 0)),
            pl.BlockSpec((D, H), lambda i: (0, 0)),
            pl.BlockSpec((1, H), lambda i: (0, 0)),
            pl.BlockSpec((H, C), lambda i: (0, 0)),
            pl.BlockSpec((1, C), lambda i: (0, 0)),
            pl.BlockSpec((br, 1), lambda i: (i, 0)),
            pl.BlockSpec((br, 1), lambda i: (i, 0)),
        ],
        out_specs=[
            pl.BlockSpec((br, C), lambda i: (i, 0)),
            pl.BlockSpec((br, C), lambda i: (i, 0)),
            pl.BlockSpec((br, C), lambda i: (i, 0)),
        ],
        out_shape=[
            jax.ShapeDtypeStruct((NPAD, C), jnp.float32),
            jax.ShapeDtypeStruct((NPAD, C), jnp.float32),
            jax.ShapeDtypeStruct((NPAD, C), jnp.float32),
        ],
    )(x, w1t, b1r, w2t, b2r, d0, d1)


def _final_body(u_ref, d0_ref, d1_ref, o_ref):
    deg = d0_ref[...] + d1_ref[...] + 1.0
    ob = jnp.sqrt(deg) * u_ref[...]
    mx = jnp.max(ob, axis=1, keepdims=True)
    e = jnp.exp(ob - mx)
    ssum = jnp.sum(e, axis=1, keepdims=True)
    o_ref[...] = ob - mx - jnp.log(ssum)


def _final(u, d0, d1):
    br = 4000
    return pl.pallas_call(
        _final_body,
        grid=(N // br,),
        in_specs=[
            pl.BlockSpec((br, C), lambda i: (i, 0)),
            pl.BlockSpec((br, 1), lambda i: (i, 0)),
            pl.BlockSpec((br, 1), lambda i: (i, 0)),
        ],
        out_specs=pl.BlockSpec((br, C), lambda i: (i, 0)),
        out_shape=jax.ShapeDtypeStruct((N, C), jnp.float32),
    )(u, d0, d1)


def kernel(x, edge_index, W1, b1, W2, b2):
    ei = edge_index.astype(jnp.int32)
    srcp = ei[0].reshape(NW, NCH, CB)
    dstp = ei[1].reshape(NW, NCH, CB)
    dstc = ei[1].reshape(NW, NCHC, CBC)

    ones_h = jnp.ones((CBC,), jnp.float32)
    zeros1 = jnp.zeros((RPT,), jnp.float32)

    pdeg = _count(dstc, ones_h, zeros1)
    d0 = pdeg[0].reshape(NPAD, 1)
    d1 = pdeg[1].reshape(NPAD, 1)

    u0, c1b, c2b = _proj(x, W1.T, b1.reshape(1, H), W2.T, b2.reshape(1, C),
                         d0, d1)

    u = u0
    for _ in range(K):
        p = _round(u, srcp, dstp)
        u = _combine(p, u, c1b, c2b)

    return _final(u, d0, d1)


# speedup vs baseline: 1.0779x; 1.0779x over previous
import functools

import jax
import jax.numpy as jnp
from jax import lax
from jax.experimental import pallas as pl
from jax.experimental.pallas import tpu as pltpu
from jax.experimental.pallas import tpu_sc as plsc

N = 100000
D = 128
H = 64
C = 16
K = 10
ALPHA = 0.1

NC = 2
NS = 16
NW = NC * NS

CB = 512
NCH = 197
EPW = NCH * CB
EPAD = NW * EPW

CBC = 10000
NCHC = 10

NPAD = 100352
RPT = NPAD // NS
RPW = NPAD // NW
CRC = RPW // 4
ZR = 392

_mesh = plsc.VectorSubcoreMesh(core_axis_name="c", subcore_axis_name="s")


@functools.partial(
    pl.kernel,
    out_type=jax.ShapeDtypeStruct((NC, NPAD), jnp.float32),
    mesh=_mesh,
    compiler_params=pltpu.CompilerParams(use_tc_tiling_on_sc=False),
    scratch_types=[
        pltpu.VMEM((3, CBC), jnp.int32),
        pltpu.VMEM((CBC,), jnp.float32),
        pltpu.VMEM((RPT,), jnp.float32),
        pltpu.VMEM_SHARED((NPAD,), jnp.float32),
        pltpu.SemaphoreType.DMA,
        pltpu.SemaphoreType.DMA,
        pltpu.SemaphoreType.DMA,
        pltpu.SemaphoreType.DMA,
        pltpu.SemaphoreType.DMA,
    ],
)
def _count(dstp, ones_h, zeros_h, deg_out, idx, ones_v, zb, agg,
           ia, ib, ic, sa, sb):
    c = lax.axis_index("c")
    s = lax.axis_index("s")
    w = c * NS + s
    isem = [ia, ib, ic]
    ssem = [sa, sb]
    pltpu.async_copy(dstp.at[w, 0], idx.at[0], isem[0])
    pltpu.async_copy(dstp.at[w, 1], idx.at[1], isem[1])
    pltpu.sync_copy(zeros_h, zb)
    pltpu.sync_copy(ones_h, ones_v)
    pltpu.sync_copy(zb, agg.at[pl.ds(s * RPT, RPT)])
    plsc.subcore_barrier()

    for ch in range(NCHC):
        pltpu.make_async_copy(dstp.at[0, 0], idx.at[ch % 3],
                              isem[ch % 3]).wait()
        if ch >= 1:
            pltpu.make_async_copy(ones_v, agg.at[pl.ds(0, CBC)],
                                  ssem[(ch - 1) % 2]).wait()
        pltpu.async_copy(ones_v, agg.at[idx.at[ch % 3]], ssem[ch % 2],
                         add=True)
        if ch + 2 < NCHC:
            pltpu.async_copy(dstp.at[w, ch + 2], idx.at[(ch + 2) % 3],
                             isem[(ch + 2) % 3])
    pltpu.make_async_copy(ones_v, agg.at[pl.ds(0, CBC)],
                          ssem[(NCHC - 1) % 2]).wait()
    plsc.subcore_barrier()
    pltpu.sync_copy(agg.at[pl.ds(s * RPT, RPT)],
                    deg_out.at[c, pl.ds(s * RPT, RPT)])


@functools.partial(
    pl.kernel,
    out_type=jax.ShapeDtypeStruct((NC, NPAD, C), jnp.float32),
    mesh=_mesh,
    compiler_params=pltpu.CompilerParams(use_tc_tiling_on_sc=False),
    scratch_types=[
        pltpu.VMEM((4, 2, CB), jnp.int32),
        pltpu.VMEM((3, CB, C), jnp.float32),
        pltpu.VMEM_SHARED((NPAD, C), jnp.float32),
        pltpu.SemaphoreType.DMA,
        pltpu.SemaphoreType.DMA,
        pltpu.SemaphoreType.DMA,
        pltpu.SemaphoreType.DMA,
        pltpu.SemaphoreType.DMA,
        pltpu.SemaphoreType.DMA,
        pltpu.SemaphoreType.DMA,
        pltpu.SemaphoreType.DMA,
        pltpu.SemaphoreType.DMA,
        pltpu.SemaphoreType.DMA,
        pltpu.SemaphoreType.DMA,
    ],
)
def _round(u, eip, p_out, idx, rows, agg,
           i0, i1, i2, i3, g0, g1, g2, s0, s1, s2, zs):
    c = lax.axis_index("c")
    s = lax.axis_index("s")
    w = c * NS + s
    isem = [i0, i1, i2, i3]
    gsem = [g0, g1, g2]
    ssem = [s0, s1, s2]

    def i_start(ch, b):
        pltpu.async_copy(eip.at[w, ch], idx.at[b], isem[b])

    def i_wait(b):
        pltpu.make_async_copy(eip.at[0, 0], idx.at[b], isem[b]).wait()

    def g_start(ib, b):
        pltpu.async_copy(u.at[idx.at[ib].at[0]], rows.at[b], gsem[b])

    def g_wait(b):
        pltpu.make_async_copy(u.at[pl.ds(0, CB)], rows.at[b], gsem[b]).wait()

    def s_start(ib, b):
        pltpu.async_copy(rows.at[b], agg.at[idx.at[ib].at[1]], ssem[b],
                         add=True)

    def s_wait(b):
        pltpu.make_async_copy(rows.at[b], agg.at[pl.ds(0, CB)],
                              ssem[b]).wait()

    i_start(0, 0)
    i_start(1, 1)
    i_start(2, 2)
    i_wait(0)
    g_start(0, 0)
    i_wait(1)
    g_start(1, 1)

    def zfill(j, carry):
        rows[2, j] = jnp.zeros((C,), jnp.float32)
        return carry

    lax.fori_loop(0, CB, zfill, 0)
    nz = RPT // CB
    for j in range(nz):
        pltpu.async_copy(rows.at[2], agg.at[pl.ds(s * RPT + j * CB, CB)], zs)
    rem = RPT - nz * CB
    if rem:
        pltpu.async_copy(rows.at[2].at[pl.ds(0, rem)],
                         agg.at[pl.ds(s * RPT + nz * CB, rem)], zs)
    for j in range(nz):
        pltpu.make_async_copy(rows.at[2], agg.at[pl.ds(0, CB)], zs).wait()
    if rem:
        pltpu.make_async_copy(rows.at[2].at[pl.ds(0, rem)],
                              agg.at[pl.ds(0, rem)], zs).wait()
    plsc.subcore_barrier()

    def step(ch, j):
        r = j % 3
        g_wait(r)
        s_start(j % 4, r)

        @pl.when(ch >= 1)
        def _():
            s_wait((j + 2) % 3)

        i_wait((j + 2) % 4)
        g_start((j + 2) % 4, (j + 2) % 3)

        @pl.when(ch + 3 < NCH)
        def _():
            i_start(ch + 3, (j + 3) % 4)

    def twelve(m, carry):
        for j in range(12):
            step(m * 12 + j, j)
        return carry

    nfull = (NCH - 2) // 12
    lax.fori_loop(0, nfull, twelve, 0)
    for t in range(NCH - 2 - nfull * 12):
        step(nfull * 12 + t, t)

    e = NCH - 2
    g_wait(e % 3)
    s_wait((e + 2) % 3)
    s_start(e % 4, e % 3)
    e = NCH - 1
    g_wait(e % 3)
    s_start(e % 4, e % 3)
    s_wait((NCH - 2) % 3)
    s_wait((NCH - 1) % 3)

    plsc.subcore_barrier()
    pltpu.sync_copy(agg.at[pl.ds(s * RPT, RPT)],
                    p_out.at[c, pl.ds(s * RPT, RPT)])


NTC = 7
CR2 = RPW // NTC


@functools.partial(
    pl.kernel,
    out_type=jax.ShapeDtypeStruct((NPAD, C), jnp.float32),
    mesh=_mesh,
    compiler_params=pltpu.CompilerParams(use_tc_tiling_on_sc=False),
    scratch_types=[
        pltpu.VMEM((2, CR2, C), jnp.float32),
        pltpu.VMEM((2, CR2, C), jnp.float32),
        pltpu.VMEM((2, CR2, C), jnp.float32),
        pltpu.VMEM((2, CR2, C), jnp.float32),
        pltpu.VMEM((2, CR2, C), jnp.float32),
        pltpu.SemaphoreType.DMA,
        pltpu.SemaphoreType.DMA,
        pltpu.SemaphoreType.DMA,
        pltpu.SemaphoreType.DMA,
    ],
)
def _combine(p, u, c1b, c2b, un, bu, b0, b1, bc1, bc2, la, lb, wa, wb):
    c = lax.axis_index("c")
    s = lax.axis_index("s")
    w = c * NS + s
    base = w * RPW
    lsem = [la, lb]
    wsem = [wa, wb]

    def loads(t, b):
        r0 = base + t * CR2
        pltpu.async_copy(u.at[pl.ds(r0, CR2)], bu.at[b], lsem[b])
        pltpu.async_copy(p.at[0, pl.ds(r0, CR2)], b0.at[b], lsem[b])
        pltpu.async_copy(p.at[1, pl.ds(r0, CR2)], b1.at[b], lsem[b])
        pltpu.async_copy(c1b.at[pl.ds(r0, CR2)], bc1.at[b], lsem[b])
        pltpu.async_copy(c2b.at[pl.ds(r0, CR2)], bc2.at[b], lsem[b])

    def loads_wait(b):
        for ref in (bu, b0, b1, bc1, bc2):
            pltpu.make_async_copy(u.at[pl.ds(0, CR2)], ref.at[b],
                                  lsem[b]).wait()

    def store(t, b):
        pltpu.async_copy(bu.at[b], un.at[pl.ds(base + t * CR2, CR2)],
                         wsem[b])

    def store_wait(b):
        pltpu.make_async_copy(bu.at[b], un.at[pl.ds(base, CR2)],
                              wsem[b]).wait()

    loads(0, 0)
    for t in range(NTC):
        b = t % 2
        nb = (t + 1) % 2
        loads_wait(b)
        if t + 1 < NTC:
            if t >= 1:
                store_wait(nb)
            loads(t + 1, nb)

        def row(j, carry):
            for q in range(4):
                i = j * 4 + q
                bu[b, i] = bc1[b, i] * (b0[b, i] + b1[b, i] + bu[b, i]) \
                    + bc2[b, i]
            return carry

        lax.fori_loop(0, CR2 // 4, row, 0)
        store(t, b)
    store_wait((NTC - 2) % 2)
    store_wait((NTC - 1) % 2)


def _proj_body(x_ref, w1_ref, b1_ref, w2_ref, b2_ref, d0_ref, d1_ref,
               u0_ref, c1_ref, c2_ref):
    i = pl.program_id(0)
    xb = x_ref[...]
    h = jnp.maximum(
        jnp.dot(xb, w1_ref[...], preferred_element_type=jnp.float32)
        + b1_ref[...], 0.0)
    zb = (jnp.dot(h, w2_ref[...], preferred_element_type=jnp.float32)
          + b2_ref[...])
    deg = d0_ref[...] + d1_ref[...] + 1.0
    rows = lax.broadcasted_iota(jnp.int32, deg.shape, 0) + i * deg.shape[0]
    m = jnp.where(rows < N, 1.0, 0.0)
    dinv = lax.rsqrt(deg)
    u0 = zb * dinv * m
    u0_ref[...] = u0
    c1_ref[...] = jnp.broadcast_to((1.0 - ALPHA) / deg * m, zb.shape)
    c2_ref[...] = ALPHA * u0


def _proj(x, w1t, b1r, w2t, b2r, d0, d1):
    br = 4096
    grid = ((NPAD + br - 1) // br,)
    return pl.pallas_call(
        _proj_body,
        grid=grid,
        in_specs=[
            pl.BlockSpec((br, D), lambda i: (i, 0)),
            pl.BlockSpec((D, H), lambda i: (0, 0)),
            pl.BlockSpec((1, H), lambda i: (0, 0)),
            pl.BlockSpec((H, C), lambda i: (0, 0)),
            pl.BlockSpec((1, C), lambda i: (0, 0)),
            pl.BlockSpec((br, 1), lambda i: (i, 0)),
            pl.BlockSpec((br, 1), lambda i: (i, 0)),
        ],
        out_specs=[
            pl.BlockSpec((br, C), lambda i: (i, 0)),
            pl.BlockSpec((br, C), lambda i: (i, 0)),
            pl.BlockSpec((br, C), lambda i: (i, 0)),
        ],
        out_shape=[
            jax.ShapeDtypeStruct((NPAD, C), jnp.float32),
            jax.ShapeDtypeStruct((NPAD, C), jnp.float32),
            jax.ShapeDtypeStruct((NPAD, C), jnp.float32),
        ],
    )(x, w1t, b1r, w2t, b2r, d0, d1)


def _final_body(u_ref, d0_ref, d1_ref, o_ref):
    deg = d0_ref[...] + d1_ref[...] + 1.0
    ob = jnp.sqrt(deg) * u_ref[...]
    mx = jnp.max(ob, axis=1, keepdims=True)
    e = jnp.exp(ob - mx)
    ssum = jnp.sum(e, axis=1, keepdims=True)
    o_ref[...] = ob - mx - jnp.log(ssum)


def _final(u, d0, d1):
    br = 4000
    return pl.pallas_call(
        _final_body,
        grid=(N // br,),
        in_specs=[
            pl.BlockSpec((br, C), lambda i: (i, 0)),
            pl.BlockSpec((br, 1), lambda i: (i, 0)),
            pl.BlockSpec((br, 1), lambda i: (i, 0)),
        ],
        out_specs=pl.BlockSpec((br, C), lambda i: (i, 0)),
        out_shape=jax.ShapeDtypeStruct((N, C), jnp.float32),
    )(u, d0, d1)


def kernel(x, edge_index, W1, b1, W2, b2):
    ei = edge_index.astype(jnp.int32)
    e = ei.shape[1]
    padn = EPAD - e
    pad_idx = N + (jnp.arange(padn, dtype=jnp.int32) % (NPAD - N))
    src_r = jnp.concatenate([ei[0], pad_idx]).reshape(NW, NCH, CB)
    dst_r = jnp.concatenate([ei[1], pad_idx]).reshape(NW, NCH, CB)
    eip = jnp.stack([src_r, dst_r], axis=2)
    dstc = ei[1].reshape(NW, NCHC, CBC)

    ones_h = jnp.ones((CBC,), jnp.float32)
    zeros1 = jnp.zeros((RPT,), jnp.float32)

    pdeg = _count(dstc, ones_h, zeros1)
    d0 = pdeg[0].reshape(NPAD, 1)
    d1 = pdeg[1].reshape(NPAD, 1)

    u0, c1b, c2b = _proj(x, W1.T, b1.reshape(1, H), W2.T, b2.reshape(1, C),
                         d0, d1)

    u = u0
    for _ in range(K):
        p = _round(u, eip)
        u = _combine(p, u, c1b, c2b)

    return _final(u, d0, d1)
